# Initial kernel scaffold; baseline (speedup 1.0000x reference)
import jax
import jax.numpy as jnp
from jax.experimental import pallas as pl


def kernel(x, edge_index, etype, bases0, coeff0, bias0, bases1, coeff1, bias1, bases2, coeff2, bias2, gate_w, gate_b):
    raise NotImplementedError("write your pallas kernel here")


# speedup vs baseline: 40.3060x; 40.3060x over previous
import functools

import jax
import jax.numpy as jnp
from jax import lax
from jax.experimental import pallas as pl
from jax.experimental.pallas import tpu as pltpu
from jax.experimental.pallas import tpu_sc as plsc

N, E, D, R, NB = 10000, 320000, 128, 8, 8

NC, NS = 2, 16
NW = NC * NS
EW = E // NW
SUB = 125
ROWS_W = EW // SUB
IG = 16
NG = ROWS_W // IG
RPT = 624
TAIL = N - NS * RPT
ZR = 16
BN = 2000


def _sc_edge_body(table, gidx, dstr, out, gidx_v, dst_v, rows_v, hacc, sem):
    c = lax.axis_index("c")
    s = lax.axis_index("s")
    wid = s * NC + c

    zvec = jnp.zeros((16,), jnp.float32)

    def zbuf(i, carry):
        for j in range(D // 16):
            rows_v[0, i, pl.ds(j * 16, 16)] = zvec
        return carry

    lax.fori_loop(0, ZR, zbuf, 0)
    zero_v = rows_v.at[0, pl.ds(0, ZR)]

    def zacc(k, carry):
        pltpu.sync_copy(zero_v, hacc.at[pl.ds(s * RPT + k * ZR, ZR)])
        return carry

    lax.fori_loop(0, RPT // ZR, zacc, 0)

    @pl.when(s == NS - 1)
    def _():
        pltpu.sync_copy(zero_v, hacc.at[pl.ds(NS * RPT, TAIL)])

    plsc.subcore_barrier()

    def group(g, carry):
        pltpu.sync_copy(gidx.at[wid, pl.ds(g * IG, IG)], gidx_v)
        pltpu.sync_copy(dstr.at[wid, pl.ds(g * IG, IG)], dst_v)
        prev = None
        for j in range(IG):
            b = j & 1
            cur = pltpu.async_copy(table.at[gidx_v.at[j]], rows_v.at[b], sem)
            if prev is not None:
                pcp, pb, pj = prev
                pcp.wait()
                pltpu.sync_copy(rows_v.at[pb], hacc.at[dst_v.at[pj]],
                                add=True)
            prev = (cur, b, j)
        pcp, pb, pj = prev
        pcp.wait()
        pltpu.sync_copy(rows_v.at[pb], hacc.at[dst_v.at[pj]], add=True)
        return carry

    lax.fori_loop(0, NG, group, 0)
    plsc.subcore_barrier()

    pltpu.sync_copy(hacc.at[pl.ds(s * RPT, RPT)],
                    out.at[c, pl.ds(s * RPT, RPT)])

    @pl.when(s == NS - 1)
    def _():
        pltpu.sync_copy(hacc.at[pl.ds(NS * RPT, TAIL)],
                        out.at[c, pl.ds(NS * RPT, TAIL)])


_SC_EDGE_CACHE = []


def _sc_edge(table, gidx, dstr):
    if not _SC_EDGE_CACHE:
        _SC_EDGE_CACHE.append(functools.partial(
            pl.kernel,
            mesh=plsc.VectorSubcoreMesh(
                core_axis_name="c", subcore_axis_name="s", num_cores=NC),
            out_type=jax.ShapeDtypeStruct((NC, N, D), jnp.float32),
            scratch_types=[
                pltpu.VMEM((IG, SUB), jnp.int32),
                pltpu.VMEM((IG, SUB), jnp.int32),
                pltpu.VMEM((2, SUB, D), jnp.float32),
                pltpu.VMEM_SHARED((N, D), jnp.float32),
                pltpu.SemaphoreType.DMA,
            ],
        )(_sc_edge_body))
    return _SC_EDGE_CACHE[0](table, gidx, dstr)


def _wr(bases_ref, coeff_ref, r):
    w = coeff_ref[r, 0] * bases_ref[0]
    for b in range(1, NB):
        w = w + coeff_ref[r, b] * bases_ref[b]
    return w


def _tc_first_body(x_ref, bases_ref, coeff_ref, out_ref):
    r = pl.program_id(1)
    a = x_ref[...]
    w = _wr(bases_ref, coeff_ref, r)
    out_ref[...] = jnp.dot(a, w, preferred_element_type=jnp.float32)


def _tc_mid_body(p0_ref, p1_ref, bias_ref, bases_ref, coeff_ref, out_ref):
    r = pl.program_id(1)
    a = jnp.maximum(p0_ref[...] + p1_ref[...] + bias_ref[...], 0.0)
    w = _wr(bases_ref, coeff_ref, r)
    out_ref[...] = jnp.dot(a, w, preferred_element_type=jnp.float32)


_ACT_SPECS = [
    pl.BlockSpec((NB, D, D), lambda i, r: (0, 0, 0)),
    pl.BlockSpec((R, NB), lambda i, r: (0, 0)),
]
_OUT_SPEC = pl.BlockSpec((BN, D), lambda i, r: (r * (N // BN) + i, 0))


def _tc_first(x, bases, coeff):
    return pl.pallas_call(
        _tc_first_body,
        grid=(N // BN, R),
        in_specs=[pl.BlockSpec((BN, D), lambda i, r: (i, 0))] + _ACT_SPECS,
        out_specs=_OUT_SPEC,
        out_shape=jax.ShapeDtypeStruct((R * N, D), jnp.float32),
    )(x, bases, coeff)


def _tc_mid(p0, p1, bias, bases, coeff):
    return pl.pallas_call(
        _tc_mid_body,
        grid=(N // BN, R),
        in_specs=[
            pl.BlockSpec((BN, D), lambda i, r: (i, 0)),
            pl.BlockSpec((BN, D), lambda i, r: (i, 0)),
            pl.BlockSpec((1, D), lambda i, r: (0, 0)),
        ] + _ACT_SPECS,
        out_specs=_OUT_SPEC,
        out_shape=jax.ShapeDtypeStruct((R * N, D), jnp.float32),
    )(p0, p1, bias, bases, coeff)


def _pool_body(p0_ref, p1_ref, bias_ref, gw_ref, gb_ref, out_ref,
               m_ref, den_ref, num_ref):
    ph = pl.program_id(0)
    j = pl.program_id(1)
    nb = pl.num_programs(1)
    h = p0_ref[...] + p1_ref[...] + bias_ref[...]
    g = jnp.sum(h * gw_ref[...], axis=1, keepdims=True) + gb_ref[...]

    @pl.when(ph == 0)
    def _():
        bm = jnp.max(g)

        @pl.when(j == 0)
        def _():
            m_ref[0] = bm

        @pl.when(j > 0)
        def _():
            m_ref[0] = jnp.maximum(m_ref[0], bm)

    @pl.when(ph == 1)
    def _():
        e = jnp.exp(g - m_ref[0])
        bnum = jnp.sum(h * e, axis=0, keepdims=True)
        bden = jnp.sum(e)

        @pl.when(j == 0)
        def _():
            num_ref[...] = bnum
            den_ref[0] = bden

        @pl.when(j > 0)
        def _():
            num_ref[...] = num_ref[...] + bnum
            den_ref[0] = den_ref[0] + bden

        @pl.when(j == nb - 1)
        def _():
            out_ref[...] = num_ref[...] / den_ref[0]


def _pool(p0, p1, bias, gw_row, gb):
    return pl.pallas_call(
        _pool_body,
        grid=(2, N // BN),
        in_specs=[
            pl.BlockSpec((BN, D), lambda ph, j: (j, 0)),
            pl.BlockSpec((BN, D), lambda ph, j: (j, 0)),
            pl.BlockSpec((1, D), lambda ph, j: (0, 0)),
            pl.BlockSpec((1, D), lambda ph, j: (0, 0)),
            pl.BlockSpec((1, 1), lambda ph, j: (0, 0)),
        ],
        out_specs=pl.BlockSpec((1, D), lambda ph, j: (0, 0)),
        out_shape=jax.ShapeDtypeStruct((1, D), jnp.float32),
        scratch_shapes=[
            pltpu.SMEM((1,), jnp.float32),
            pltpu.SMEM((1,), jnp.float32),
            pltpu.VMEM((1, D), jnp.float32),
        ],
    )(p0, p1, bias, gw_row, gb)


def kernel(x, edge_index, etype, bases0, coeff0, bias0, bases1, coeff1, bias1,
           bases2, coeff2, bias2, gate_w, gate_b):
    src = edge_index[0]
    dst = edge_index[1]
    gidx = (etype * N + src).reshape(NW, ROWS_W, SUB)
    dstr = dst.reshape(NW, ROWS_W, SUB)
    b0 = bias0.reshape(1, D)
    b1 = bias1.reshape(1, D)
    b2 = bias2.reshape(1, D)

    t0 = _tc_first(x, bases0, coeff0)
    parts = _sc_edge(t0, gidx, dstr)
    t1 = _tc_mid(parts[0], parts[1], b0, bases1, coeff1)
    parts = _sc_edge(t1, gidx, dstr)
    t2 = _tc_mid(parts[0], parts[1], b1, bases2, coeff2)
    parts = _sc_edge(t2, gidx, dstr)
    return _pool(parts[0], parts[1], b2, gate_w.reshape(1, D),
                 gate_b.reshape(1, 1))
